# baseline (speedup 1.0000x reference)
import functools
import math

import jax
import jax.numpy as jnp
import numpy as np
from jax import lax
from jax.experimental import pallas as pl
from jax.experimental.pallas import tpu as pltpu
from jax.experimental.pallas import tpu_sc as plsc

N = 10000
D = 256
E = 160000
ED = 64
HID = 128
V = 5
H = 4

NP = 10240
EPAD = 163840
EPT = EPAD // 32
NP4 = NP * 4

_SQ6 = math.sqrt(6.0)


def _gelu(x):
    return 0.5 * x * (1.0 + lax.erf(x * 0.7071067811865476))


def _lnrow(y, s, b):
    mu = jnp.mean(y, axis=-1, keepdims=True)
    var = jnp.mean((y - mu) ** 2, axis=-1, keepdims=True)
    return (y - mu) * lax.rsqrt(var + 1e-5) * s + b


def _leaky(z):
    return jnp.where(z > 0, z, 0.2 * z)


_BN = 1000


def _tc1_body(x_ref, wm_ref, gatb_ref, a_ref, w1ab_ref, b1_ref,
              g1w_ref, p_ref, adst_ref, mask_ref,
              hm_ref, xab_ref, h1_ref, den_ref, y_ref):
    x = x_ref[...]
    hm_ref[...] = jnp.dot(x, wm_ref[...], preferred_element_type=jnp.float32) + gatb_ref[...]
    asrc = jnp.dot(x, a_ref[...], preferred_element_type=jnp.float32)
    xab_ref[...] = jnp.dot(x, w1ab_ref[...], preferred_element_type=jnp.float32) + b1_ref[...]
    h1_ref[...] = jnp.dot(x, g1w_ref[...], preferred_element_type=jnp.float32)
    s = jnp.dot(asrc, p_ref[...], preferred_element_type=jnp.float32)
    s = _leaky(s + adst_ref[...])
    e = jnp.exp(s) * mask_ref[...]

    @pl.when(pl.program_id(0) == 0)
    def _():
        den_ref[...] = jnp.zeros_like(den_ref)
        y_ref[...] = jnp.zeros_like(y_ref)

    den_ref[...] += jnp.sum(e, axis=0, keepdims=True)
    y_ref[...] += lax.dot_general(e, x, (((0,), (0,)), ((), ())),
                                  preferred_element_type=jnp.float32)


def _tc1(x, Wm, gat_b, A32, W1ab, b1, g1W, P, adst_f, mask_f):
    g = N // _BN
    c = lambda shape: pl.BlockSpec(shape, lambda i: (0, 0))
    r = lambda w: pl.BlockSpec((_BN, w), lambda i: (i, 0))
    return pl.pallas_call(
        _tc1_body,
        grid=(g,),
        in_specs=[r(D), c((D, D)), c((1, D)), c((D, 32)), c((D, 2 * ED)),
                  c((1, 2 * ED)), c((D, D)), c((32, 32)), c((1, 32)),
                  c((1, 32))],
        out_specs=[r(D),
                   pl.BlockSpec((_BN, 2 * ED), lambda i: (i, 0)),
                   r(D), c((8, 32)), c((32, D))],
        out_shape=[
            jax.ShapeDtypeStruct((N, D), jnp.float32),
            jax.ShapeDtypeStruct((NP, 2 * ED), jnp.float32),
            jax.ShapeDtypeStruct((N, D), jnp.float32),
            jax.ShapeDtypeStruct((8, 32), jnp.float32),
            jax.ShapeDtypeStruct((32, D), jnp.float32),
        ],
        compiler_params=pltpu.CompilerParams(
            dimension_semantics=("arbitrary",)),
    )(x, Wm, gat_b, A32, W1ab, b1, g1W, P, adst_f, mask_f)


def _vfin_body(y_ref, den_ref, eself_ref, hv_ref, gatw_ref, gatb_ref,
               l1s_ref, l1b_ref, gcnw_ref, gcnb_ref, out_ref):
    dent = den_ref[0:1, :] + eself_ref[...]
    acc = jnp.zeros((1, D), jnp.float32)
    for j in range(V):
        gj = jnp.zeros((1, D), jnp.float32)
        for h in range(H):
            k = j * H + h
            num = jnp.dot(y_ref[k:k + 1, :], gatw_ref[:, h * D:(h + 1) * D],
                          preferred_element_type=jnp.float32)
            num = num + eself_ref[0, k] * hv_ref[k:k + 1, :]
            gj = gj + num / dent[0, k]
        gj = gj * (1.0 / H) + gatb_ref[...]
        gj = _lnrow(_gelu(gj), l1s_ref[...], l1b_ref[...])
        acc = acc + gj
    svec = jnp.dot(acc, gcnw_ref[...], preferred_element_type=jnp.float32)
    svec = svec * (1.0 / _SQ6) + gcnb_ref[...]
    out_ref[...] = jnp.broadcast_to(svec, (8, D))


def _vfin(Y, den, eself_f, hv_f, gat_W, gat_b, l1s, l1b, gcn_W, gcn_b):
    c = lambda shape: pl.BlockSpec(shape, lambda: tuple(0 for _ in shape))
    return pl.pallas_call(
        _vfin_body,
        in_specs=[c((32, D)), c((8, 32)), c((1, 32)), c((32, D)),
                  c((D, H * D)), c((1, D)), c((1, D)), c((1, D)),
                  c((D, D)), c((1, D))],
        out_specs=c((8, D)),
        out_shape=jax.ShapeDtypeStruct((8, D), jnp.float32),
    )(Y, den, eself_f, hv_f, gat_W, gat_b, l1s, l1b, gcn_W, gcn_b)


_CHG = 160


def _scg_body(row_ref, col_ref, ta_ref, tb_ref, outa_ref, outb_ref,
              idxa, idxb, bufa0, bufb0, bufa1, bufb1,
              semg0, semg1, semw0, semw1):
    cid = lax.axis_index("c")
    sid = lax.axis_index("s")
    wid = sid * 2 + cid
    base = wid * EPT
    CH = _CHG
    nch = EPT // CH
    bufs = ((bufa0, bufb0, semg0, semw0),
            (bufa1, bufb1, semg1, semw1))

    pltpu.sync_copy(row_ref.at[pl.ds(base, EPT)], idxa)
    pltpu.sync_copy(col_ref.at[pl.ds(base, EPT)], idxb)

    def issue(i, b):
        ba, bb, sg, _ = bufs[b]

        @pl.when(i < nch)
        def _():
            off = i * CH
            pltpu.async_copy(ta_ref.at[idxa.at[pl.ds(off, CH)]], ba, sg)
            pltpu.async_copy(tb_ref.at[idxb.at[pl.ds(off, CH)]], bb, sg)

    issue(jnp.int32(0), 0)
    issue(jnp.int32(1), 1)

    def step(i2, _):
        for b in range(2):
            i = i2 * 2 + b
            ba, bb, sg, sw = bufs[b]
            off = base + i * CH
            pltpu.make_async_copy(ta_ref.at[idxa.at[pl.ds(0, CH)]], ba, sg).wait()
            pltpu.make_async_copy(tb_ref.at[idxb.at[pl.ds(0, CH)]], bb, sg).wait()
            pltpu.async_copy(ba, outa_ref.at[pl.ds(off, CH)], sw)
            pltpu.async_copy(bb, outb_ref.at[pl.ds(off, CH)], sw)
            pltpu.make_async_copy(ba, outa_ref.at[pl.ds(0, CH)], sw).wait()
            pltpu.make_async_copy(bb, outb_ref.at[pl.ds(0, CH)], sw).wait()
            issue(i + 2, b)
        return 0

    lax.fori_loop(0, nch // 2, step, 0)


def _sc_gather(row, col, ta, tb):
    mesh = plsc.VectorSubcoreMesh(core_axis_name="c", subcore_axis_name="s")
    kfn = pl.kernel(
        _scg_body,
        out_type=[jax.ShapeDtypeStruct((EPAD, HID), jnp.float32),
                  jax.ShapeDtypeStruct((EPAD, HID), jnp.float32)],
        mesh=mesh,
        scratch_types=[
            pltpu.VMEM((EPT,), jnp.int32),
            pltpu.VMEM((EPT,), jnp.int32),
            pltpu.VMEM((_CHG, HID), jnp.float32),
            pltpu.VMEM((_CHG, HID), jnp.float32),
            pltpu.VMEM((_CHG, HID), jnp.float32),
            pltpu.VMEM((_CHG, HID), jnp.float32),
            pltpu.SemaphoreType.DMA,
            pltpu.SemaphoreType.DMA,
            pltpu.SemaphoreType.DMA,
            pltpu.SemaphoreType.DMA,
        ],
        compiler_params=pltpu.CompilerParams(needs_layout_passes=False),
    )
    return kfn(row, col, ta, tb)


def _scgs_body(row_ref, col_ref, ta_ref, tb_ref, out_ref,
               idxa, idxb, bufa0, bufb0, bufa1, bufb1,
               semg0, semg1, semw0, semw1):
    cid = lax.axis_index("c")
    sid = lax.axis_index("s")
    wid = sid * 2 + cid
    base = wid * EPT
    CH = _CHG
    nch = EPT // CH
    bufs = ((bufa0, bufb0, semg0, semw0),
            (bufa1, bufb1, semg1, semw1))

    pltpu.sync_copy(row_ref.at[pl.ds(base, EPT)], idxa)
    pltpu.sync_copy(col_ref.at[pl.ds(base, EPT)], idxb)

    def issue(i, b):
        ba, bb, sg, _ = bufs[b]

        @pl.when(i < nch)
        def _():
            off = i * CH
            pltpu.async_copy(ta_ref.at[idxa.at[pl.ds(off, CH)]], ba, sg)
            pltpu.async_copy(tb_ref.at[idxb.at[pl.ds(off, CH)]], bb, sg)

    issue(jnp.int32(0), 0)
    issue(jnp.int32(1), 1)

    def step(i2, _):
        for b in range(2):
            i = i2 * 2 + b
            ba, bb, sg, sw = bufs[b]
            off = base + i * CH
            pltpu.make_async_copy(ta_ref.at[idxa.at[pl.ds(0, CH)]], ba, sg).wait()
            pltpu.make_async_copy(tb_ref.at[idxb.at[pl.ds(0, CH)]], bb, sg).wait()

            @plsc.parallel_loop(0, CH, 1, unroll=2)
            def _(r):
                for k in range(HID // 16):
                    ba[r, pl.ds(k * 16, 16)] = (ba[r, pl.ds(k * 16, 16)]
                                                + bb[r, pl.ds(k * 16, 16)])

            pltpu.async_copy(ba, out_ref.at[pl.ds(off, CH)], sw)
            pltpu.make_async_copy(ba, out_ref.at[pl.ds(0, CH)], sw).wait()
            issue(i + 2, b)
        return 0

    lax.fori_loop(0, nch // 2, step, 0)


def _sc_gather_sum(row, col, ta, tb):
    mesh = plsc.VectorSubcoreMesh(core_axis_name="c", subcore_axis_name="s")
    kfn = pl.kernel(
        _scgs_body,
        out_type=jax.ShapeDtypeStruct((EPAD, HID), jnp.float32),
        mesh=mesh,
        scratch_types=[
            pltpu.VMEM((EPT,), jnp.int32),
            pltpu.VMEM((EPT,), jnp.int32),
            pltpu.VMEM((_CHG, HID), jnp.float32),
            pltpu.VMEM((_CHG, HID), jnp.float32),
            pltpu.VMEM((_CHG, HID), jnp.float32),
            pltpu.VMEM((_CHG, HID), jnp.float32),
            pltpu.SemaphoreType.DMA,
            pltpu.SemaphoreType.DMA,
            pltpu.SemaphoreType.DMA,
            pltpu.SemaphoreType.DMA,
        ],
        compiler_params=pltpu.CompilerParams(needs_layout_passes=False),
    )
    return kfn(row, col, ta, tb)


def _deg_body(col_ref, w_ref, out_ref, accv, cv, wv):
    cid = lax.axis_index("c")
    sid = lax.axis_index("s")
    wid = sid * 2 + cid
    base = wid * EPT

    @plsc.parallel_loop(0, NP // 16, 1, unroll=8)
    def _(i):
        accv[pl.ds(i * 16, 16)] = jnp.zeros((16,), jnp.float32)

    pltpu.sync_copy(col_ref.at[pl.ds(base, EPT)], cv)
    pltpu.sync_copy(w_ref.at[pl.ds(base, EPT)], wv)

    @plsc.parallel_loop(0, EPT // 16, 1, unroll=4)
    def _(k):
        c16 = cv[pl.ds(k * 16, 16)]
        w16 = wv[pl.ds(k * 16, 16)]
        plsc.addupdate_scatter(accv, [c16], w16)

    pltpu.sync_copy(accv, out_ref.at[wid])


def _sc_deg(col, w):
    mesh = plsc.VectorSubcoreMesh(core_axis_name="c", subcore_axis_name="s")
    kfn = pl.kernel(
        _deg_body,
        out_type=jax.ShapeDtypeStruct((32, NP), jnp.float32),
        mesh=mesh,
        scratch_types=[
            pltpu.VMEM((NP,), jnp.float32),
            pltpu.VMEM((EPT,), jnp.int32),
            pltpu.VMEM((EPT,), jnp.float32),
        ],
        compiler_params=pltpu.CompilerParams(needs_layout_passes=False),
    )
    return kfn(col, w)


_C2 = 10240


def _agg_body(rc_ref, w_ref, tab_ref, out_ref, tabv, accv,
              pv0, wv0, pv1, wv1, sem0, sem1):
    cid = lax.axis_index("c")
    sid = lax.axis_index("s")
    wid = sid * 2 + cid
    nch = EPAD // _C2
    bufs = ((pv0, wv0, sem0), (pv1, wv1, sem1))

    def issue(g, b):
        pv, wv, sem = bufs[b]

        @pl.when(g < nch)
        def _():
            off = g * _C2
            pltpu.async_copy(rc_ref.at[pl.ds(off, _C2)], pv, sem)
            pltpu.async_copy(w_ref.at[pl.ds(off, _C2)], wv, sem)

    def drain(b):
        pv, wv, sem = bufs[b]
        pltpu.make_async_copy(rc_ref.at[pl.ds(0, _C2)], pv, sem).wait()
        pltpu.make_async_copy(w_ref.at[pl.ds(0, _C2)], wv, sem).wait()

    for p in range(2):
        q = p * 32 + wid
        pltpu.sync_copy(tab_ref.at[q], tabv)

        @plsc.parallel_loop(0, NP4 // 16, 1, unroll=8)
        def _(i):
            accv[pl.ds(i * 16, 16)] = jnp.zeros((16,), jnp.float32)

        issue(jnp.int32(0), 0)
        issue(jnp.int32(1), 1)

        def chunk(g2, _):
            for b in range(2):
                g = g2 * 2 + b
                pv, wv, sem = bufs[b]
                drain(b)

                @plsc.parallel_loop(0, _C2 // 16, 1, unroll=6)
                def _(k):
                    p16 = pv[pl.ds(k * 16, 16)]
                    w16 = wv[pl.ds(k * 16, 16)]
                    r16 = p16 & 0xFFFF
                    c16 = (p16 >> 16) * 4
                    for f in range(4):
                        v = plsc.load_gather(tabv, [r16 + f])
                        plsc.addupdate_scatter(accv, [c16 + f], v * w16)

                issue(g + 2, b)
            return 0

        lax.fori_loop(0, nch // 2, chunk, 0)
        pltpu.sync_copy(accv, out_ref.at[q])


def _sc_agg(rc, w, tab):
    mesh = plsc.VectorSubcoreMesh(core_axis_name="c", subcore_axis_name="s")
    kfn = pl.kernel(
        _agg_body,
        out_type=jax.ShapeDtypeStruct((64, NP4), jnp.float32),
        mesh=mesh,
        scratch_types=[
            pltpu.VMEM((NP4,), jnp.float32),
            pltpu.VMEM((NP4,), jnp.float32),
            pltpu.VMEM((_C2,), jnp.int32),
            pltpu.VMEM((_C2,), jnp.float32),
            pltpu.VMEM((_C2,), jnp.int32),
            pltpu.VMEM((_C2,), jnp.float32),
            pltpu.SemaphoreType.DMA,
            pltpu.SemaphoreType.DMA,
        ],
        compiler_params=pltpu.CompilerParams(needs_layout_passes=False),
    )
    return kfn(rc, w, tab)


_BE = 2048


def _tc3_body(e0a_ref, e0b_ref, w2_ref, b2_ref, ewt_ref, ewb_ref,
              ef_ref, w1_ref):
    ef0 = jnp.maximum(e0a_ref[:, :ED] + e0b_ref[:, ED:], 0.0)
    ef = jnp.dot(ef0, w2_ref[...], preferred_element_type=jnp.float32) + b2_ref[...]
    ef_ref[...] = ef
    logit = lax.dot_general(ewt_ref[...], ef, (((1,), (1,)), ((), ())),
                            preferred_element_type=jnp.float32) + ewb_ref[...]
    w1_ref[...] = jax.nn.sigmoid(logit)[None]


def _tc3(e0a, e0b, W2, b2, ewT, ewb):
    g = EPAD // _BE
    c = lambda shape: pl.BlockSpec(shape, lambda i: (0, 0))
    return pl.pallas_call(
        _tc3_body,
        grid=(g,),
        in_specs=[pl.BlockSpec((_BE, 2 * ED), lambda i: (i, 0)),
                  pl.BlockSpec((_BE, 2 * ED), lambda i: (i, 0)),
                  c((ED, ED)), c((1, ED)), c((1, ED)), c((1, 1))],
        out_specs=[pl.BlockSpec((_BE, ED), lambda i: (i, 0)),
                   pl.BlockSpec((1, 1, _BE), lambda i: (i, 0, 0))],
        out_shape=[jax.ShapeDtypeStruct((EPAD, ED), jnp.float32),
                   jax.ShapeDtypeStruct((EPAD // _BE, 1, _BE), jnp.float32)],
    )(e0a, e0b, W2, b2, ewT, ewb)


def _tc4a_body(degp_ref, out_ref):
    s = jnp.sum(degp_ref[...], axis=0)
    out_ref[...] = lax.rsqrt(1.0 + s)


def _tc4a(degp3):
    return pl.pallas_call(
        _tc4a_body,
        in_specs=[pl.BlockSpec((32, NP // 128, 128), lambda: (0, 0, 0))],
        out_specs=pl.BlockSpec((NP // 128, 128), lambda: (0, 0)),
        out_shape=jax.ShapeDtypeStruct((NP // 128, 128), jnp.float32),
    )(degp3)


def _tc4b_body(h_ref, dis_ref, out_ref):
    out_ref[...] = h_ref[...] * dis_ref[...]


def _tc4b(h, dis_col):
    g = N // _BN
    return pl.pallas_call(
        _tc4b_body,
        grid=(g,),
        in_specs=[pl.BlockSpec((_BN, D), lambda i: (i, 0)),
                  pl.BlockSpec((_BN, 1), lambda i: (i, 0))],
        out_specs=pl.BlockSpec((_BN, D), lambda i: (i, 0)),
        out_shape=jax.ShapeDtypeStruct((N, D), jnp.float32),
    )(h, dis_col)


def _tc5_body(s1_ref, h1d_ref, dis_ref, b_ref, l1s_ref, l1b_ref,
              w1a_ref, eb1_ref, w1b_ref, g2w_ref, ua_ref, ub_ref, h2_ref):
    o1pre = dis_ref[...] * (s1_ref[...] + h1d_ref[...]) + b_ref[...]
    o1 = _lnrow(_gelu(o1pre), l1s_ref[...], l1b_ref[...])
    ua_ref[...] = jnp.dot(o1, w1a_ref[...], preferred_element_type=jnp.float32) + eb1_ref[...]
    ub_ref[...] = jnp.dot(o1, w1b_ref[...], preferred_element_type=jnp.float32)
    h2_ref[...] = jnp.dot(o1, g2w_ref[...], preferred_element_type=jnp.float32)


def _tc5(s1, h1d, dis_col, gcn1_b, l1s, l1b, euW1a, eub1, euW1b, g2W):
    g = N // _BN
    c = lambda shape: pl.BlockSpec(shape, lambda i: (0, 0))
    r = lambda w: pl.BlockSpec((_BN, w), lambda i: (i, 0))
    return pl.pallas_call(
        _tc5_body,
        grid=(g,),
        in_specs=[r(D), r(D), pl.BlockSpec((_BN, 1), lambda i: (i, 0)),
                  c((1, D)), c((1, D)), c((1, D)),
                  c((D, HID)), c((1, HID)), c((D, HID)), c((D, D))],
        out_specs=[r(HID), r(HID), r(D)],
        out_shape=[jax.ShapeDtypeStruct((NP, HID), jnp.float32),
                   jax.ShapeDtypeStruct((NP, HID), jnp.float32),
                   jax.ShapeDtypeStruct((N, D), jnp.float32)],
    )(s1, h1d, dis_col, gcn1_b, l1s, l1b, euW1a, eub1, euW1b, g2W)


def _tc6_body(uab_ref, ef_ref, w1e_ref, w2_ref, b2_ref,
              lns_ref, lnb_ref, ewt_ref, ewb_ref, out_ref):
    t = uab_ref[...] + jnp.dot(
        ef_ref[...], w1e_ref[...], preferred_element_type=jnp.float32)
    t = jnp.maximum(t, 0.0)
    upd = jnp.dot(t, w2_ref[...], preferred_element_type=jnp.float32) + b2_ref[...]
    y = ef_ref[...] + upd
    ly = _lnrow(y, lns_ref[...], lnb_ref[...])
    logit = lax.dot_general(ewt_ref[...], ly, (((1,), (1,)), ((), ())),
                            preferred_element_type=jnp.float32) + ewb_ref[...]
    out_ref[...] = jax.nn.sigmoid(logit)[None]


def _tc6(uab, ef, W1e, euW2, eub2, lns, lnb, ewT, ewb):
    g = EPAD // _BE
    c = lambda shape: pl.BlockSpec(shape, lambda i: (0, 0))
    return pl.pallas_call(
        _tc6_body,
        grid=(g,),
        in_specs=[pl.BlockSpec((_BE, HID), lambda i: (i, 0)),
                  pl.BlockSpec((_BE, ED), lambda i: (i, 0)),
                  c((ED, HID)), c((HID, ED)), c((1, ED)),
                  c((1, ED)), c((1, ED)), c((1, ED)), c((1, 1))],
        out_specs=pl.BlockSpec((1, 1, _BE), lambda i: (i, 0, 0)),
        out_shape=jax.ShapeDtypeStruct((EPAD // _BE, 1, _BE), jnp.float32),
    )(uab, ef, W1e, euW2, eub2, lns, lnb, ewT, ewb)


def _tc8_body(hm_ref, svec_ref, gcnw_ref, hl1s_ref, hl1b_ref, hl2s_ref,
              hl2b_ref, s2_ref, h2d_ref, dis_ref, g2b_ref, ol2s_ref,
              ol2b_ref, ew_ref, out_ref):
    g = _lnrow(_gelu(hm_ref[...]), hl1s_ref[...], hl1b_ref[...])
    hi_pre = jnp.dot(g, gcnw_ref[...], preferred_element_type=jnp.float32)
    hi_pre = hi_pre * (1.0 / 6.0) + svec_ref[0:1, :]
    hi = _lnrow(_gelu(hi_pre), hl2s_ref[...], hl2b_ref[...])
    o2pre = dis_ref[...] * (s2_ref[...] + h2d_ref[...]) + g2b_ref[...]
    o2 = _lnrow(_gelu(o2pre), ol2s_ref[...], ol2b_ref[...])
    ew = ew_ref[...]
    out_ref[...] = ew[:, 0:1] * hi + ew[:, 1:2] * o2


def _tc8(hm, svec, gcn_W, hl1s, hl1b, hl2s, hl2b, s2, h2d, dis_col,
         gcn2_b, ol2s, ol2b, ew):
    g = N // _BN
    c = lambda shape: pl.BlockSpec(shape, lambda i: (0, 0))
    r = lambda w: pl.BlockSpec((_BN, w), lambda i: (i, 0))
    return pl.pallas_call(
        _tc8_body,
        grid=(g,),
        in_specs=[r(D), c((8, D)), c((D, D)), c((1, D)), c((1, D)),
                  c((1, D)), c((1, D)), r(D), r(D),
                  pl.BlockSpec((_BN, 1), lambda i: (i, 0)),
                  c((1, D)), c((1, D)), c((1, D)),
                  pl.BlockSpec((_BN, 2), lambda i: (i, 0))],
        out_specs=r(D),
        out_shape=jax.ShapeDtypeStruct((N, D), jnp.float32),
    )(hm, svec, gcn_W, hl1s, hl1b, hl2s, hl2b, s2, h2d, dis_col,
      gcn2_b, ol2s, ol2b, ew)


def _row2(v):
    return v.reshape(1, -1)


def kernel(x, edge_index, expert_weights, params):
    ho, oh = params["ho"], params["oh"]

    gat_W = ho["gat_W"]
    W3 = gat_W.reshape(D, H, D)
    Wm = W3.mean(1)
    A = jnp.einsum("dhe,he->dh", W3, ho["att_src"])
    A32 = jnp.pad(A, ((0, 0), (0, 32 - H)))
    hv = (ho["vn"] @ gat_W).reshape(V, H, D)
    asrcv = (hv * ho["att_src"][None]).sum(-1)
    adstv = (hv * ho["att_dst"][None]).sum(-1)
    hv_f = jnp.pad(hv.reshape(V * H, D), ((0, 32 - V * H), (0, 0)))
    P = np.zeros((32, 32), np.float32)
    for j in range(V):
        for h in range(H):
            P[h, j * H + h] = 1.0
    P = jnp.asarray(P)
    adst_f = jnp.pad(adstv.reshape(1, V * H), ((0, 0), (0, 32 - V * H)))
    mask_f = jnp.asarray(
        np.pad(np.ones((1, V * H), np.float32), ((0, 0), (0, 32 - V * H))))

    W1ab = jnp.concatenate([oh["ei_W1"][:D], oh["ei_W1"][D:]], axis=1)
    b1ab = jnp.concatenate([oh["ei_b1"], jnp.zeros((ED,), jnp.float32)])
    euW1a = oh["eu_W1"][:D]
    euW1b = oh["eu_W1"][D:2 * D]
    W1e = oh["eu_W1"][2 * D:]
    ewT = _row2(oh["ew_W"][:, 0])
    ewb = oh["ew_b"].reshape(1, 1)

    dump = N + (jnp.arange(EPAD - E, dtype=jnp.int32) % (NP - N))
    row = jnp.concatenate([edge_index[0], dump])
    col = jnp.concatenate([edge_index[1], dump])
    row4 = row * 4
    rcpack = row4 | (col << 16)

    eself = jnp.exp(_leaky(asrcv + adstv))
    eself_f = jnp.pad(eself.reshape(1, V * H), ((0, 0), (0, 32 - V * H)))
    hm, xab_p, h1, den, Y = _tc1(
        x, Wm, _row2(ho["gat_b"]), A32, W1ab, _row2(b1ab), oh["gcn1_W"],
        P, adst_f, mask_f)

    svec = _vfin(Y, den, eself_f, hv_f, gat_W, _row2(ho["gat_b"]),
                 _row2(ho["ln1_s"]), _row2(ho["ln1_b"]), ho["gcn_W"],
                 _row2(ho["gcn_b"]))

    e0a, e0b = _sc_gather(row, col, xab_p, xab_p)
    ef, w1 = _tc3(e0a, e0b, oh["ei_W2"], _row2(oh["ei_b2"]), ewT, ewb)
    w1f = w1.reshape(EPAD)

    degp1 = _sc_deg(col, w1f)
    dis1 = _tc4a(degp1.reshape(32, NP // 128, 128)).reshape(NP)
    dis1c = dis1[:N].reshape(N, 1)
    h1d = _tc4b(h1, dis1c)
    h1t = jnp.pad(h1d, ((0, NP - N), (0, 0))).reshape(NP, 64, 4)
    h1t = h1t.transpose(1, 0, 2).reshape(64, NP4)
    s1q = _sc_agg(rcpack, w1f, h1t)
    s1 = s1q.reshape(64, NP, 4).transpose(1, 0, 2).reshape(NP, D)[:N]

    ua, ub, h2 = _tc5(s1, h1d, dis1c, _row2(oh["gcn1_b"]),
                      _row2(oh["ln1_s"]), _row2(oh["ln1_b"]),
                      euW1a, _row2(oh["eu_b1"]), euW1b, oh["gcn2_W"])

    uab = _sc_gather_sum(row, col, ua, ub)
    w2 = _tc6(uab, ef, W1e, oh["eu_W2"], _row2(oh["eu_b2"]),
              _row2(oh["lne_s"]), _row2(oh["lne_b"]), ewT, ewb)
    w2f = w2.reshape(EPAD)

    degp2 = _sc_deg(col, w2f)
    dis2 = _tc4a(degp2.reshape(32, NP // 128, 128)).reshape(NP)
    dis2c = dis2[:N].reshape(N, 1)
    h2d = _tc4b(h2, dis2c)
    h2t = jnp.pad(h2d, ((0, NP - N), (0, 0))).reshape(NP, 64, 4)
    h2t = h2t.transpose(1, 0, 2).reshape(64, NP4)
    s2q = _sc_agg(rcpack, w2f, h2t)
    s2 = s2q.reshape(64, NP, 4).transpose(1, 0, 2).reshape(NP, D)[:N]

    return _tc8(hm, svec, ho["gcn_W"], _row2(ho["ln1_s"]),
                _row2(ho["ln1_b"]), _row2(ho["ln2_s"]), _row2(ho["ln2_b"]),
                s2, h2d, dis2c, _row2(oh["gcn2_b"]), _row2(oh["ln2_s"]),
                _row2(oh["ln2_b"]), expert_weights)


# speedup vs baseline: 1.0151x; 1.0151x over previous
import functools
import math

import jax
import jax.numpy as jnp
import numpy as np
from jax import lax
from jax.experimental import pallas as pl
from jax.experimental.pallas import tpu as pltpu
from jax.experimental.pallas import tpu_sc as plsc

N = 10000
D = 256
E = 160000
ED = 64
HID = 128
V = 5
H = 4

NP = 10240
EPAD = 163840
EPT = EPAD // 32
NP4 = NP * 4

_SQ6 = math.sqrt(6.0)


def _gelu(x):
    return 0.5 * x * (1.0 + lax.erf(x * 0.7071067811865476))


def _lnrow(y, s, b):
    mu = jnp.mean(y, axis=-1, keepdims=True)
    var = jnp.mean((y - mu) ** 2, axis=-1, keepdims=True)
    return (y - mu) * lax.rsqrt(var + 1e-5) * s + b


def _leaky(z):
    return jnp.where(z > 0, z, 0.2 * z)


_BN = 1000


def _tc1_body(x_ref, wm_ref, gatb_ref, a_ref, w1ab_ref, b1_ref,
              g1w_ref, p_ref, adst_ref, mask_ref,
              hm_ref, xab_ref, h1_ref, den_ref, y_ref):
    x = x_ref[...]
    hm_ref[...] = jnp.dot(x, wm_ref[...], preferred_element_type=jnp.float32) + gatb_ref[...]
    asrc = jnp.dot(x, a_ref[...], preferred_element_type=jnp.float32)
    xab_ref[...] = jnp.dot(x, w1ab_ref[...], preferred_element_type=jnp.float32) + b1_ref[...]
    h1_ref[...] = jnp.dot(x, g1w_ref[...], preferred_element_type=jnp.float32)
    s = jnp.dot(asrc, p_ref[...], preferred_element_type=jnp.float32)
    s = _leaky(s + adst_ref[...])
    e = jnp.exp(s) * mask_ref[...]

    @pl.when(pl.program_id(0) == 0)
    def _():
        den_ref[...] = jnp.zeros_like(den_ref)
        y_ref[...] = jnp.zeros_like(y_ref)

    den_ref[...] += jnp.sum(e, axis=0, keepdims=True)
    y_ref[...] += lax.dot_general(e, x, (((0,), (0,)), ((), ())),
                                  preferred_element_type=jnp.float32)


def _tc1(x, Wm, gat_b, A32, W1ab, b1, g1W, P, adst_f, mask_f):
    g = N // _BN
    c = lambda shape: pl.BlockSpec(shape, lambda i: (0, 0))
    r = lambda w: pl.BlockSpec((_BN, w), lambda i: (i, 0))
    return pl.pallas_call(
        _tc1_body,
        grid=(g,),
        in_specs=[r(D), c((D, D)), c((1, D)), c((D, 32)), c((D, 2 * ED)),
                  c((1, 2 * ED)), c((D, D)), c((32, 32)), c((1, 32)),
                  c((1, 32))],
        out_specs=[r(D),
                   pl.BlockSpec((_BN, 2 * ED), lambda i: (i, 0)),
                   r(D), c((8, 32)), c((32, D))],
        out_shape=[
            jax.ShapeDtypeStruct((N, D), jnp.float32),
            jax.ShapeDtypeStruct((NP, 2 * ED), jnp.float32),
            jax.ShapeDtypeStruct((N, D), jnp.float32),
            jax.ShapeDtypeStruct((8, 32), jnp.float32),
            jax.ShapeDtypeStruct((32, D), jnp.float32),
        ],
        compiler_params=pltpu.CompilerParams(
            dimension_semantics=("arbitrary",)),
    )(x, Wm, gat_b, A32, W1ab, b1, g1W, P, adst_f, mask_f)


def _vfin_body(y_ref, den_ref, eself_ref, hv_ref, gatw_ref, gatb_ref,
               l1s_ref, l1b_ref, gcnw_ref, gcnb_ref, out_ref):
    dent = den_ref[0:1, :] + eself_ref[...]
    acc = jnp.zeros((1, D), jnp.float32)
    for j in range(V):
        gj = jnp.zeros((1, D), jnp.float32)
        for h in range(H):
            k = j * H + h
            num = jnp.dot(y_ref[k:k + 1, :], gatw_ref[:, h * D:(h + 1) * D],
                          preferred_element_type=jnp.float32)
            num = num + eself_ref[0, k] * hv_ref[k:k + 1, :]
            gj = gj + num / dent[0, k]
        gj = gj * (1.0 / H) + gatb_ref[...]
        gj = _lnrow(_gelu(gj), l1s_ref[...], l1b_ref[...])
        acc = acc + gj
    svec = jnp.dot(acc, gcnw_ref[...], preferred_element_type=jnp.float32)
    svec = svec * (1.0 / _SQ6) + gcnb_ref[...]
    out_ref[...] = jnp.broadcast_to(svec, (8, D))


def _vfin(Y, den, eself_f, hv_f, gat_W, gat_b, l1s, l1b, gcn_W, gcn_b):
    c = lambda shape: pl.BlockSpec(shape, lambda: tuple(0 for _ in shape))
    return pl.pallas_call(
        _vfin_body,
        in_specs=[c((32, D)), c((8, 32)), c((1, 32)), c((32, D)),
                  c((D, H * D)), c((1, D)), c((1, D)), c((1, D)),
                  c((D, D)), c((1, D))],
        out_specs=c((8, D)),
        out_shape=jax.ShapeDtypeStruct((8, D), jnp.float32),
    )(Y, den, eself_f, hv_f, gat_W, gat_b, l1s, l1b, gcn_W, gcn_b)


_CHG = 160


def _scg_body(row_ref, col_ref, ta_ref, tb_ref, outa_ref, outb_ref,
              idxa, idxb, bufa0, bufb0, bufa1, bufb1,
              semg0, semg1, semw0, semw1):
    cid = lax.axis_index("c")
    sid = lax.axis_index("s")
    wid = sid * 2 + cid
    base = wid * EPT
    CH = _CHG
    nch = EPT // CH
    bufs = ((bufa0, bufb0, semg0, semw0),
            (bufa1, bufb1, semg1, semw1))

    pltpu.sync_copy(row_ref.at[pl.ds(base, EPT)], idxa)
    pltpu.sync_copy(col_ref.at[pl.ds(base, EPT)], idxb)

    def issue(i, b):
        ba, bb, sg, _ = bufs[b]

        @pl.when(i < nch)
        def _():
            off = i * CH
            pltpu.async_copy(ta_ref.at[idxa.at[pl.ds(off, CH)]], ba, sg)
            pltpu.async_copy(tb_ref.at[idxb.at[pl.ds(off, CH)]], bb, sg)

    issue(jnp.int32(0), 0)
    issue(jnp.int32(1), 1)

    def step(i2, _):
        for b in range(2):
            i = i2 * 2 + b
            ba, bb, sg, sw = bufs[b]
            off = base + i * CH
            pltpu.make_async_copy(ta_ref.at[idxa.at[pl.ds(0, CH)]], ba, sg).wait()
            pltpu.make_async_copy(tb_ref.at[idxb.at[pl.ds(0, CH)]], bb, sg).wait()
            pltpu.async_copy(ba, outa_ref.at[pl.ds(off, CH)], sw)
            pltpu.async_copy(bb, outb_ref.at[pl.ds(off, CH)], sw)
            pltpu.make_async_copy(ba, outa_ref.at[pl.ds(0, CH)], sw).wait()
            pltpu.make_async_copy(bb, outb_ref.at[pl.ds(0, CH)], sw).wait()
            issue(i + 2, b)
        return 0

    lax.fori_loop(0, nch // 2, step, 0)


def _sc_gather(row, col, ta, tb):
    mesh = plsc.VectorSubcoreMesh(core_axis_name="c", subcore_axis_name="s")
    kfn = pl.kernel(
        _scg_body,
        out_type=[jax.ShapeDtypeStruct((EPAD, HID), jnp.float32),
                  jax.ShapeDtypeStruct((EPAD, HID), jnp.float32)],
        mesh=mesh,
        scratch_types=[
            pltpu.VMEM((EPT,), jnp.int32),
            pltpu.VMEM((EPT,), jnp.int32),
            pltpu.VMEM((_CHG, HID), jnp.float32),
            pltpu.VMEM((_CHG, HID), jnp.float32),
            pltpu.VMEM((_CHG, HID), jnp.float32),
            pltpu.VMEM((_CHG, HID), jnp.float32),
            pltpu.SemaphoreType.DMA,
            pltpu.SemaphoreType.DMA,
            pltpu.SemaphoreType.DMA,
            pltpu.SemaphoreType.DMA,
        ],
        compiler_params=pltpu.CompilerParams(needs_layout_passes=False),
    )
    return kfn(row, col, ta, tb)


def _scgs_body(row_ref, col_ref, ta_ref, tb_ref, out_ref,
               idxa, idxb, bufa0, bufb0, bufa1, bufb1,
               semg0, semg1, semw0, semw1):
    cid = lax.axis_index("c")
    sid = lax.axis_index("s")
    wid = sid * 2 + cid
    base = wid * EPT
    CH = _CHG
    nch = EPT // CH
    bufs = ((bufa0, bufb0, semg0, semw0),
            (bufa1, bufb1, semg1, semw1))

    pltpu.sync_copy(row_ref.at[pl.ds(base, EPT)], idxa)
    pltpu.sync_copy(col_ref.at[pl.ds(base, EPT)], idxb)

    def issue(i, b):
        ba, bb, sg, _ = bufs[b]

        @pl.when(i < nch)
        def _():
            off = i * CH
            pltpu.async_copy(ta_ref.at[idxa.at[pl.ds(off, CH)]], ba, sg)
            pltpu.async_copy(tb_ref.at[idxb.at[pl.ds(off, CH)]], bb, sg)

    issue(jnp.int32(0), 0)
    issue(jnp.int32(1), 1)

    def step(i2, _):
        for b in range(2):
            i = i2 * 2 + b
            ba, bb, sg, sw = bufs[b]
            off = base + i * CH
            pltpu.make_async_copy(ta_ref.at[idxa.at[pl.ds(0, CH)]], ba, sg).wait()
            pltpu.make_async_copy(tb_ref.at[idxb.at[pl.ds(0, CH)]], bb, sg).wait()

            @plsc.parallel_loop(0, CH, 1, unroll=2)
            def _(r):
                for k in range(HID // 16):
                    ba[r, pl.ds(k * 16, 16)] = (ba[r, pl.ds(k * 16, 16)]
                                                + bb[r, pl.ds(k * 16, 16)])

            pltpu.async_copy(ba, out_ref.at[pl.ds(off, CH)], sw)
            pltpu.make_async_copy(ba, out_ref.at[pl.ds(0, CH)], sw).wait()
            issue(i + 2, b)
        return 0

    lax.fori_loop(0, nch // 2, step, 0)


def _sc_gather_sum(row, col, ta, tb):
    mesh = plsc.VectorSubcoreMesh(core_axis_name="c", subcore_axis_name="s")
    kfn = pl.kernel(
        _scgs_body,
        out_type=jax.ShapeDtypeStruct((EPAD, HID), jnp.float32),
        mesh=mesh,
        scratch_types=[
            pltpu.VMEM((EPT,), jnp.int32),
            pltpu.VMEM((EPT,), jnp.int32),
            pltpu.VMEM((_CHG, HID), jnp.float32),
            pltpu.VMEM((_CHG, HID), jnp.float32),
            pltpu.VMEM((_CHG, HID), jnp.float32),
            pltpu.VMEM((_CHG, HID), jnp.float32),
            pltpu.SemaphoreType.DMA,
            pltpu.SemaphoreType.DMA,
            pltpu.SemaphoreType.DMA,
            pltpu.SemaphoreType.DMA,
        ],
        compiler_params=pltpu.CompilerParams(needs_layout_passes=False),
    )
    return kfn(row, col, ta, tb)


def _deg_body(col_ref, w_ref, out_ref, accv, cv, wv):
    cid = lax.axis_index("c")
    sid = lax.axis_index("s")
    wid = sid * 2 + cid
    base = wid * EPT

    @plsc.parallel_loop(0, NP // 16, 1, unroll=8)
    def _(i):
        accv[pl.ds(i * 16, 16)] = jnp.zeros((16,), jnp.float32)

    pltpu.sync_copy(col_ref.at[pl.ds(base, EPT)], cv)
    pltpu.sync_copy(w_ref.at[pl.ds(base, EPT)], wv)

    @plsc.parallel_loop(0, EPT // 16, 1, unroll=4)
    def _(k):
        c16 = cv[pl.ds(k * 16, 16)]
        w16 = wv[pl.ds(k * 16, 16)]
        plsc.addupdate_scatter(accv, [c16], w16)

    pltpu.sync_copy(accv, out_ref.at[wid])


def _sc_deg(col, w):
    mesh = plsc.VectorSubcoreMesh(core_axis_name="c", subcore_axis_name="s")
    kfn = pl.kernel(
        _deg_body,
        out_type=jax.ShapeDtypeStruct((32, NP), jnp.float32),
        mesh=mesh,
        scratch_types=[
            pltpu.VMEM((NP,), jnp.float32),
            pltpu.VMEM((EPT,), jnp.int32),
            pltpu.VMEM((EPT,), jnp.float32),
        ],
        compiler_params=pltpu.CompilerParams(needs_layout_passes=False),
    )
    return kfn(col, w)


_C2 = 5120


def _agg_body(rc_ref, w_ref, tab_ref, out_ref, tabv, accv,
              pv0, wv0, pv1, wv1, sem0, sem1):
    cid = lax.axis_index("c")
    sid = lax.axis_index("s")
    wid = sid * 2 + cid
    nch = EPAD // _C2
    bufs = ((pv0, wv0, sem0), (pv1, wv1, sem1))

    def issue(g, b):
        pv, wv, sem = bufs[b]

        @pl.when(g < nch)
        def _():
            off = g * _C2
            pltpu.async_copy(rc_ref.at[pl.ds(off, _C2)], pv, sem)
            pltpu.async_copy(w_ref.at[pl.ds(off, _C2)], wv, sem)

    def drain(b):
        pv, wv, sem = bufs[b]
        pltpu.make_async_copy(rc_ref.at[pl.ds(0, _C2)], pv, sem).wait()
        pltpu.make_async_copy(w_ref.at[pl.ds(0, _C2)], wv, sem).wait()

    for p in range(2):
        q = p * 32 + wid
        pltpu.sync_copy(tab_ref.at[q], tabv)

        @plsc.parallel_loop(0, NP4 // 16, 1, unroll=8)
        def _(i):
            accv[pl.ds(i * 16, 16)] = jnp.zeros((16,), jnp.float32)

        issue(jnp.int32(0), 0)
        issue(jnp.int32(1), 1)

        def chunk(g2, _):
            for b in range(2):
                g = g2 * 2 + b
                pv, wv, sem = bufs[b]
                drain(b)

                @plsc.parallel_loop(0, _C2 // 16, 1, unroll=4)
                def _(k):
                    p16 = pv[pl.ds(k * 16, 16)]
                    w16 = wv[pl.ds(k * 16, 16)]
                    r16 = p16 & 0xFFFF
                    c16 = (p16 >> 16) * 4
                    for f in range(4):
                        v = plsc.load_gather(tabv, [r16 + f])
                        plsc.addupdate_scatter(accv, [c16 + f], v * w16)

                issue(g + 2, b)
            return 0

        lax.fori_loop(0, nch // 2, chunk, 0)
        pltpu.sync_copy(accv, out_ref.at[q])


def _sc_agg(rc, w, tab):
    mesh = plsc.VectorSubcoreMesh(core_axis_name="c", subcore_axis_name="s")
    kfn = pl.kernel(
        _agg_body,
        out_type=jax.ShapeDtypeStruct((64, NP4), jnp.float32),
        mesh=mesh,
        scratch_types=[
            pltpu.VMEM((NP4,), jnp.float32),
            pltpu.VMEM((NP4,), jnp.float32),
            pltpu.VMEM((_C2,), jnp.int32),
            pltpu.VMEM((_C2,), jnp.float32),
            pltpu.VMEM((_C2,), jnp.int32),
            pltpu.VMEM((_C2,), jnp.float32),
            pltpu.SemaphoreType.DMA,
            pltpu.SemaphoreType.DMA,
        ],
        compiler_params=pltpu.CompilerParams(needs_layout_passes=False),
    )
    return kfn(rc, w, tab)


_BE = 2048


def _tc3_body(e0a_ref, e0b_ref, w2_ref, b2_ref, ewt_ref, ewb_ref,
              ef_ref, w1_ref):
    ef0 = jnp.maximum(e0a_ref[:, :ED] + e0b_ref[:, ED:], 0.0)
    ef = jnp.dot(ef0, w2_ref[...], preferred_element_type=jnp.float32) + b2_ref[...]
    ef_ref[...] = ef
    logit = lax.dot_general(ewt_ref[...], ef, (((1,), (1,)), ((), ())),
                            preferred_element_type=jnp.float32) + ewb_ref[...]
    w1_ref[...] = jax.nn.sigmoid(logit)[None]


def _tc3(e0a, e0b, W2, b2, ewT, ewb):
    g = EPAD // _BE
    c = lambda shape: pl.BlockSpec(shape, lambda i: (0, 0))
    return pl.pallas_call(
        _tc3_body,
        grid=(g,),
        in_specs=[pl.BlockSpec((_BE, 2 * ED), lambda i: (i, 0)),
                  pl.BlockSpec((_BE, 2 * ED), lambda i: (i, 0)),
                  c((ED, ED)), c((1, ED)), c((1, ED)), c((1, 1))],
        out_specs=[pl.BlockSpec((_BE, ED), lambda i: (i, 0)),
                   pl.BlockSpec((1, 1, _BE), lambda i: (i, 0, 0))],
        out_shape=[jax.ShapeDtypeStruct((EPAD, ED), jnp.float32),
                   jax.ShapeDtypeStruct((EPAD // _BE, 1, _BE), jnp.float32)],
    )(e0a, e0b, W2, b2, ewT, ewb)


def _tc4a_body(degp_ref, out_ref):
    s = jnp.sum(degp_ref[...], axis=0)
    out_ref[...] = lax.rsqrt(1.0 + s)


def _tc4a(degp3):
    return pl.pallas_call(
        _tc4a_body,
        in_specs=[pl.BlockSpec((32, NP // 128, 128), lambda: (0, 0, 0))],
        out_specs=pl.BlockSpec((NP // 128, 128), lambda: (0, 0)),
        out_shape=jax.ShapeDtypeStruct((NP // 128, 128), jnp.float32),
    )(degp3)


def _tc4b_body(h_ref, dis_ref, out_ref):
    out_ref[...] = h_ref[...] * dis_ref[...]


def _tc4b(h, dis_col):
    g = N // _BN
    return pl.pallas_call(
        _tc4b_body,
        grid=(g,),
        in_specs=[pl.BlockSpec((_BN, D), lambda i: (i, 0)),
                  pl.BlockSpec((_BN, 1), lambda i: (i, 0))],
        out_specs=pl.BlockSpec((_BN, D), lambda i: (i, 0)),
        out_shape=jax.ShapeDtypeStruct((N, D), jnp.float32),
    )(h, dis_col)


def _tc5_body(s1_ref, h1d_ref, dis_ref, b_ref, l1s_ref, l1b_ref,
              w1a_ref, eb1_ref, w1b_ref, g2w_ref, ua_ref, ub_ref, h2_ref):
    o1pre = dis_ref[...] * (s1_ref[...] + h1d_ref[...]) + b_ref[...]
    o1 = _lnrow(_gelu(o1pre), l1s_ref[...], l1b_ref[...])
    ua_ref[...] = jnp.dot(o1, w1a_ref[...], preferred_element_type=jnp.float32) + eb1_ref[...]
    ub_ref[...] = jnp.dot(o1, w1b_ref[...], preferred_element_type=jnp.float32)
    h2_ref[...] = jnp.dot(o1, g2w_ref[...], preferred_element_type=jnp.float32)


def _tc5(s1, h1d, dis_col, gcn1_b, l1s, l1b, euW1a, eub1, euW1b, g2W):
    g = N // _BN
    c = lambda shape: pl.BlockSpec(shape, lambda i: (0, 0))
    r = lambda w: pl.BlockSpec((_BN, w), lambda i: (i, 0))
    return pl.pallas_call(
        _tc5_body,
        grid=(g,),
        in_specs=[r(D), r(D), pl.BlockSpec((_BN, 1), lambda i: (i, 0)),
                  c((1, D)), c((1, D)), c((1, D)),
                  c((D, HID)), c((1, HID)), c((D, HID)), c((D, D))],
        out_specs=[r(HID), r(HID), r(D)],
        out_shape=[jax.ShapeDtypeStruct((NP, HID), jnp.float32),
                   jax.ShapeDtypeStruct((NP, HID), jnp.float32),
                   jax.ShapeDtypeStruct((N, D), jnp.float32)],
    )(s1, h1d, dis_col, gcn1_b, l1s, l1b, euW1a, eub1, euW1b, g2W)


def _tc6_body(uab_ref, ef_ref, w1e_ref, w2_ref, b2_ref,
              lns_ref, lnb_ref, ewt_ref, ewb_ref, out_ref):
    t = uab_ref[...] + jnp.dot(
        ef_ref[...], w1e_ref[...], preferred_element_type=jnp.float32)
    t = jnp.maximum(t, 0.0)
    upd = jnp.dot(t, w2_ref[...], preferred_element_type=jnp.float32) + b2_ref[...]
    y = ef_ref[...] + upd
    ly = _lnrow(y, lns_ref[...], lnb_ref[...])
    logit = lax.dot_general(ewt_ref[...], ly, (((1,), (1,)), ((), ())),
                            preferred_element_type=jnp.float32) + ewb_ref[...]
    out_ref[...] = jax.nn.sigmoid(logit)[None]


def _tc6(uab, ef, W1e, euW2, eub2, lns, lnb, ewT, ewb):
    g = EPAD // _BE
    c = lambda shape: pl.BlockSpec(shape, lambda i: (0, 0))
    return pl.pallas_call(
        _tc6_body,
        grid=(g,),
        in_specs=[pl.BlockSpec((_BE, HID), lambda i: (i, 0)),
                  pl.BlockSpec((_BE, ED), lambda i: (i, 0)),
                  c((ED, HID)), c((HID, ED)), c((1, ED)),
                  c((1, ED)), c((1, ED)), c((1, ED)), c((1, 1))],
        out_specs=pl.BlockSpec((1, 1, _BE), lambda i: (i, 0, 0)),
        out_shape=jax.ShapeDtypeStruct((EPAD // _BE, 1, _BE), jnp.float32),
    )(uab, ef, W1e, euW2, eub2, lns, lnb, ewT, ewb)


def _tc8_body(hm_ref, svec_ref, gcnw_ref, hl1s_ref, hl1b_ref, hl2s_ref,
              hl2b_ref, s2_ref, h2d_ref, dis_ref, g2b_ref, ol2s_ref,
              ol2b_ref, ew_ref, out_ref):
    g = _lnrow(_gelu(hm_ref[...]), hl1s_ref[...], hl1b_ref[...])
    hi_pre = jnp.dot(g, gcnw_ref[...], preferred_element_type=jnp.float32)
    hi_pre = hi_pre * (1.0 / 6.0) + svec_ref[0:1, :]
    hi = _lnrow(_gelu(hi_pre), hl2s_ref[...], hl2b_ref[...])
    o2pre = dis_ref[...] * (s2_ref[...] + h2d_ref[...]) + g2b_ref[...]
    o2 = _lnrow(_gelu(o2pre), ol2s_ref[...], ol2b_ref[...])
    ew = ew_ref[...]
    out_ref[...] = ew[:, 0:1] * hi + ew[:, 1:2] * o2


def _tc8(hm, svec, gcn_W, hl1s, hl1b, hl2s, hl2b, s2, h2d, dis_col,
         gcn2_b, ol2s, ol2b, ew):
    g = N // _BN
    c = lambda shape: pl.BlockSpec(shape, lambda i: (0, 0))
    r = lambda w: pl.BlockSpec((_BN, w), lambda i: (i, 0))
    return pl.pallas_call(
        _tc8_body,
        grid=(g,),
        in_specs=[r(D), c((8, D)), c((D, D)), c((1, D)), c((1, D)),
                  c((1, D)), c((1, D)), r(D), r(D),
                  pl.BlockSpec((_BN, 1), lambda i: (i, 0)),
                  c((1, D)), c((1, D)), c((1, D)),
                  pl.BlockSpec((_BN, 2), lambda i: (i, 0))],
        out_specs=r(D),
        out_shape=jax.ShapeDtypeStruct((N, D), jnp.float32),
    )(hm, svec, gcn_W, hl1s, hl1b, hl2s, hl2b, s2, h2d, dis_col,
      gcn2_b, ol2s, ol2b, ew)


def _row2(v):
    return v.reshape(1, -1)


def kernel(x, edge_index, expert_weights, params):
    ho, oh = params["ho"], params["oh"]

    gat_W = ho["gat_W"]
    W3 = gat_W.reshape(D, H, D)
    Wm = W3.mean(1)
    A = jnp.einsum("dhe,he->dh", W3, ho["att_src"])
    A32 = jnp.pad(A, ((0, 0), (0, 32 - H)))
    hv = (ho["vn"] @ gat_W).reshape(V, H, D)
    asrcv = (hv * ho["att_src"][None]).sum(-1)
    adstv = (hv * ho["att_dst"][None]).sum(-1)
    hv_f = jnp.pad(hv.reshape(V * H, D), ((0, 32 - V * H), (0, 0)))
    P = np.zeros((32, 32), np.float32)
    for j in range(V):
        for h in range(H):
            P[h, j * H + h] = 1.0
    P = jnp.asarray(P)
    adst_f = jnp.pad(adstv.reshape(1, V * H), ((0, 0), (0, 32 - V * H)))
    mask_f = jnp.asarray(
        np.pad(np.ones((1, V * H), np.float32), ((0, 0), (0, 32 - V * H))))

    W1ab = jnp.concatenate([oh["ei_W1"][:D], oh["ei_W1"][D:]], axis=1)
    b1ab = jnp.concatenate([oh["ei_b1"], jnp.zeros((ED,), jnp.float32)])
    euW1a = oh["eu_W1"][:D]
    euW1b = oh["eu_W1"][D:2 * D]
    W1e = oh["eu_W1"][2 * D:]
    ewT = _row2(oh["ew_W"][:, 0])
    ewb = oh["ew_b"].reshape(1, 1)

    dump = N + (jnp.arange(EPAD - E, dtype=jnp.int32) % (NP - N))
    row = jnp.concatenate([edge_index[0], dump])
    col = jnp.concatenate([edge_index[1], dump])
    row4 = row * 4
    rcpack = row4 | (col << 16)

    eself = jnp.exp(_leaky(asrcv + adstv))
    eself_f = jnp.pad(eself.reshape(1, V * H), ((0, 0), (0, 32 - V * H)))
    hm, xab_p, h1, den, Y = _tc1(
        x, Wm, _row2(ho["gat_b"]), A32, W1ab, _row2(b1ab), oh["gcn1_W"],
        P, adst_f, mask_f)

    svec = _vfin(Y, den, eself_f, hv_f, gat_W, _row2(ho["gat_b"]),
                 _row2(ho["ln1_s"]), _row2(ho["ln1_b"]), ho["gcn_W"],
                 _row2(ho["gcn_b"]))

    e0a, e0b = _sc_gather(row, col, xab_p, xab_p)
    ef, w1 = _tc3(e0a, e0b, oh["ei_W2"], _row2(oh["ei_b2"]), ewT, ewb)
    w1f = w1.reshape(EPAD)

    degp1 = _sc_deg(col, w1f)
    dis1 = _tc4a(degp1.reshape(32, NP // 128, 128)).reshape(NP)
    dis1c = dis1[:N].reshape(N, 1)
    h1d = _tc4b(h1, dis1c)
    h1t = jnp.pad(h1d, ((0, NP - N), (0, 0))).reshape(NP, 64, 4)
    h1t = h1t.transpose(1, 0, 2).reshape(64, NP4)
    s1q = _sc_agg(rcpack, w1f, h1t)
    s1 = s1q.reshape(64, NP, 4).transpose(1, 0, 2).reshape(NP, D)[:N]

    ua, ub, h2 = _tc5(s1, h1d, dis1c, _row2(oh["gcn1_b"]),
                      _row2(oh["ln1_s"]), _row2(oh["ln1_b"]),
                      euW1a, _row2(oh["eu_b1"]), euW1b, oh["gcn2_W"])

    uab = _sc_gather_sum(row, col, ua, ub)
    w2 = _tc6(uab, ef, W1e, oh["eu_W2"], _row2(oh["eu_b2"]),
              _row2(oh["lne_s"]), _row2(oh["lne_b"]), ewT, ewb)
    w2f = w2.reshape(EPAD)

    degp2 = _sc_deg(col, w2f)
    dis2 = _tc4a(degp2.reshape(32, NP // 128, 128)).reshape(NP)
    dis2c = dis2[:N].reshape(N, 1)
    h2d = _tc4b(h2, dis2c)
    h2t = jnp.pad(h2d, ((0, NP - N), (0, 0))).reshape(NP, 64, 4)
    h2t = h2t.transpose(1, 0, 2).reshape(64, NP4)
    s2q = _sc_agg(rcpack, w2f, h2t)
    s2 = s2q.reshape(64, NP, 4).transpose(1, 0, 2).reshape(NP, D)[:N]

    return _tc8(hm, svec, ho["gcn_W"], _row2(ho["ln1_s"]),
                _row2(ho["ln1_b"]), _row2(ho["ln2_s"]), _row2(ho["ln2_b"]),
                s2, h2d, dis2c, _row2(oh["gcn2_b"]), _row2(oh["ln2_s"]),
                _row2(oh["ln2_b"]), expert_weights)
